# baseline (speedup 1.0000x reference)
import functools

import numpy as np
import jax
import jax.numpy as jnp
from jax.experimental import pallas as pl
from jax.experimental.pallas import tpu as pltpu


def _dct_factors():
    alpha = np.array([1.0 / np.sqrt(2)] + [1.0] * 7, dtype=np.float64)
    k = np.arange(8)
    cos = np.cos((2 * k[None, :] + 1) * k[:, None] * np.pi / 16)
    a = 0.5 * alpha[:, None] * cos
    return a


_A_NP = _dct_factors()


def _dec_kernel(ym_ref, cbm_ref, crm_ref, qy_ref, qc_ref,
                m2y_ref, m1y_ref, m2c_ref, m1c_ref, o_ref):
    cy = ym_ref[...] * qy_ref[...]
    t = jnp.dot(cy, m1y_ref[...], preferred_element_type=jnp.float32)
    yimg = jnp.dot(m2y_ref[...], t, preferred_element_type=jnp.float32) + 128.0

    ccb = cbm_ref[...] * qc_ref[...]
    ccr = crm_ref[...] * qc_ref[...]
    tcb = jnp.dot(m2c_ref[...], ccb, preferred_element_type=jnp.float32)
    tcr = jnp.dot(m2c_ref[...], ccr, preferred_element_type=jnp.float32)
    cb2 = jnp.dot(tcb, m1c_ref[...], preferred_element_type=jnp.float32)
    cr2 = jnp.dot(tcr, m1c_ref[...], preferred_element_type=jnp.float32)

    o_ref[0] = jnp.clip(yimg + 1.402 * cr2, 0.0, 255.0)
    o_ref[1] = jnp.clip(yimg - 0.344136 * cb2 - 0.714136 * cr2, 0.0, 255.0)
    o_ref[2] = jnp.clip(yimg + 1.772 * cb2, 0.0, 255.0)


def _merge_layout(x, b, nbr, nbc):
    x = x.astype(jnp.float32).reshape(b, nbr, nbc, 8, 8)
    return jnp.transpose(x, (0, 1, 3, 2, 4)).reshape(b, nbr * 8, nbc * 8)


def _decompress(y, cb, cr, y_qt, c_qt, height, width):
    b = y.shape[0]
    tile_h = 64 if height % 64 == 0 else height
    ntiles = height // tile_h
    hw, cw = width, width // 2

    ym = _merge_layout(y, b, height // 8, width // 8)
    cbm = _merge_layout(cb, b, height // 16, width // 16)
    crm = _merge_layout(cr, b, height // 16, width // 16)

    a = _A_NP
    m1y = jnp.asarray(np.kron(np.eye(width // 8), a), dtype=jnp.float32)
    m2y = jnp.asarray(np.kron(np.eye(tile_h // 8), a.T), dtype=jnp.float32)
    a_up_cols = np.repeat(a, 2, axis=1)
    a_up_rows = np.repeat(a.T, 2, axis=0)
    m1c = jnp.asarray(np.kron(np.eye(width // 16), a_up_cols),
                      dtype=jnp.float32)
    m2c = jnp.asarray(np.kron(np.eye(tile_h // 16), a_up_rows),
                      dtype=jnp.float32)

    qy = jnp.tile(y_qt.astype(jnp.float32), (tile_h // 8, width // 8))
    qc = jnp.tile(c_qt.astype(jnp.float32), (tile_h // 16, width // 16))

    cst = lambda r, c: pl.BlockSpec((r, c), lambda bi, i: (0, 0))
    return pl.pallas_call(
        _dec_kernel,
        out_shape=jax.ShapeDtypeStruct((b, 3, height, width), jnp.float32),
        grid=(b, ntiles),
        in_specs=[
            pl.BlockSpec((None, tile_h, hw), lambda bi, i: (bi, i, 0)),
            pl.BlockSpec((None, tile_h // 2, cw), lambda bi, i: (bi, i, 0)),
            pl.BlockSpec((None, tile_h // 2, cw), lambda bi, i: (bi, i, 0)),
            cst(tile_h, hw), cst(tile_h // 2, cw),
            cst(tile_h, tile_h), cst(hw, hw),
            cst(tile_h, tile_h // 2), cst(cw, hw),
        ],
        out_specs=pl.BlockSpec((None, 3, tile_h, width),
                               lambda bi, i: (bi, 0, i, 0)),
        compiler_params=pltpu.CompilerParams(
            dimension_semantics=("parallel", "arbitrary")),
    )(ym, cbm, crm, qy, qc, m2y, m1y, m2c, m1c)


def kernel(y, cb, cr, y_qt, c_qt):
    return _decompress(y, cb, cr, y_qt, c_qt, 512, 512)


# speedup vs baseline: 1.0012x; 1.0012x over previous
import functools

import numpy as np
import jax
import jax.numpy as jnp
from jax.experimental import pallas as pl
from jax.experimental.pallas import tpu as pltpu


def _dct_factors():
    alpha = np.array([1.0 / np.sqrt(2)] + [1.0] * 7, dtype=np.float64)
    k = np.arange(8)
    cos = np.cos((2 * k[None, :] + 1) * k[:, None] * np.pi / 16)
    a = 0.5 * alpha[:, None] * cos
    return a


_A_NP = _dct_factors()


def _dec_kernel(ym_ref, cbm_ref, crm_ref, qy_ref, qc_ref,
                m2y_ref, m1y_ref, m2c_ref, m1c_ref, o_ref):
    cy = ym_ref[...] * qy_ref[...]
    t = jnp.dot(cy, m1y_ref[...], preferred_element_type=jnp.float32)
    yimg = jnp.dot(m2y_ref[...], t, preferred_element_type=jnp.float32) + 128.0

    ccb = cbm_ref[...] * qc_ref[...]
    ccr = crm_ref[...] * qc_ref[...]
    tcb = jnp.dot(m2c_ref[...], ccb, preferred_element_type=jnp.float32)
    tcr = jnp.dot(m2c_ref[...], ccr, preferred_element_type=jnp.float32)
    cb2 = jnp.dot(tcb, m1c_ref[...], preferred_element_type=jnp.float32)
    cr2 = jnp.dot(tcr, m1c_ref[...], preferred_element_type=jnp.float32)

    o_ref[0] = jnp.clip(yimg + 1.402 * cr2, 0.0, 255.0)
    o_ref[1] = jnp.clip(yimg - 0.344136 * cb2 - 0.714136 * cr2, 0.0, 255.0)
    o_ref[2] = jnp.clip(yimg + 1.772 * cb2, 0.0, 255.0)


def _merge_layout(x, b, nbr, nbc):
    x = x.astype(jnp.float32).reshape(b, nbr, nbc, 8, 8)
    return jnp.transpose(x, (0, 1, 3, 2, 4)).reshape(b, nbr * 8, nbc * 8)


def _decompress(y, cb, cr, y_qt, c_qt, height, width):
    b = y.shape[0]
    tile_h = 64 if height % 64 == 0 else height
    ntiles = height // tile_h
    hw, cw = width, width // 2

    ym = _merge_layout(y, b, height // 8, width // 8)
    cbm = _merge_layout(cb, b, height // 16, width // 16)
    crm = _merge_layout(cr, b, height // 16, width // 16)

    a = _A_NP
    m1y = jnp.asarray(np.kron(np.eye(width // 8), a), dtype=jnp.float32)
    m2y = jnp.asarray(np.kron(np.eye(tile_h // 8), a.T), dtype=jnp.float32)
    a_up_cols = np.repeat(a, 2, axis=1)
    a_up_rows = np.repeat(a.T, 2, axis=0)
    m1c = jnp.asarray(np.kron(np.eye(width // 16), a_up_cols),
                      dtype=jnp.float32)
    m2c = jnp.asarray(np.kron(np.eye(tile_h // 16), a_up_rows),
                      dtype=jnp.float32)

    qy = jnp.tile(y_qt.astype(jnp.float32), (tile_h // 8, width // 8))
    qc = jnp.tile(c_qt.astype(jnp.float32), (tile_h // 16, width // 16))

    cst = lambda r, c: pl.BlockSpec((r, c), lambda bi, i: (0, 0))
    return pl.pallas_call(
        _dec_kernel,
        out_shape=jax.ShapeDtypeStruct((b, 3, height, width), jnp.float32),
        grid=(b, ntiles),
        in_specs=[
            pl.BlockSpec((None, tile_h, hw), lambda bi, i: (bi, i, 0)),
            pl.BlockSpec((None, tile_h // 2, cw), lambda bi, i: (bi, i, 0)),
            pl.BlockSpec((None, tile_h // 2, cw), lambda bi, i: (bi, i, 0)),
            cst(tile_h, hw), cst(tile_h // 2, cw),
            cst(tile_h, tile_h), cst(hw, hw),
            cst(tile_h, tile_h // 2), cst(cw, hw),
        ],
        out_specs=pl.BlockSpec((None, 3, tile_h, width),
                               lambda bi, i: (bi, 0, i, 0)),
        compiler_params=pltpu.CompilerParams(
            dimension_semantics=("parallel", "arbitrary"),
            allow_input_fusion=[True, True, True, False, False,
                                False, False, False, False]),
    )(ym, cbm, crm, qy, qc, m2y, m1y, m2c, m1c)


def kernel(y, cb, cr, y_qt, c_qt):
    return _decompress(y, cb, cr, y_qt, c_qt, 512, 512)


# speedup vs baseline: 1.2709x; 1.2694x over previous
import functools

import numpy as np
import jax
import jax.numpy as jnp
from jax.experimental import pallas as pl
from jax.experimental.pallas import tpu as pltpu


def _dct_factors():
    alpha = np.array([1.0 / np.sqrt(2)] + [1.0] * 7, dtype=np.float64)
    k = np.arange(8)
    cos = np.cos((2 * k[None, :] + 1) * k[:, None] * np.pi / 16)
    return 0.5 * alpha[:, None] * cos


_A_NP = _dct_factors()


def _merge(x, nbr, nbc):
    return (x.reshape(nbr, nbc, 8, 8).transpose(0, 2, 1, 3)
            .reshape(nbr * 8, nbc * 8))


def _dec_kernel(y_ref, cb_ref, cr_ref, qy_ref, qc_ref,
                m2y_ref, m1y_ref, m2c_ref, m1c_ref, o_ref, *,
                bry, brc, wblk, cblk):
    cy = _merge(y_ref[...].astype(jnp.bfloat16), bry, wblk)
    cy = cy.astype(jnp.float32) * qy_ref[...]
    t = jnp.dot(cy, m1y_ref[...], preferred_element_type=jnp.float32)
    yimg = jnp.dot(m2y_ref[...], t, preferred_element_type=jnp.float32) + 128.0

    ccb = _merge(cb_ref[...].astype(jnp.bfloat16), brc, cblk)
    ccr = _merge(cr_ref[...].astype(jnp.bfloat16), brc, cblk)
    ccb = ccb.astype(jnp.float32) * qc_ref[...]
    ccr = ccr.astype(jnp.float32) * qc_ref[...]
    tcb = jnp.dot(m2c_ref[...], ccb, preferred_element_type=jnp.float32)
    tcr = jnp.dot(m2c_ref[...], ccr, preferred_element_type=jnp.float32)
    cb2 = jnp.dot(tcb, m1c_ref[...], preferred_element_type=jnp.float32)
    cr2 = jnp.dot(tcr, m1c_ref[...], preferred_element_type=jnp.float32)

    o_ref[0] = jnp.clip(yimg + 1.402 * cr2, 0.0, 255.0)
    o_ref[1] = jnp.clip(yimg - 0.344136 * cb2 - 0.714136 * cr2, 0.0, 255.0)
    o_ref[2] = jnp.clip(yimg + 1.772 * cb2, 0.0, 255.0)


def _decompress(y, cb, cr, y_qt, c_qt, height, width):
    b, n_y = y.shape[0], y.shape[1]
    n_c = cb.shape[1]
    tile_h = 64 if height % 64 == 0 else height
    ntiles = height // tile_h
    wblk, cblk = width // 8, width // 16
    bry, brc = tile_h // 8, tile_h // 16

    y2 = y.astype(jnp.float32).reshape(b, n_y, 64)
    cb2 = cb.astype(jnp.float32).reshape(b, n_c, 64)
    cr2 = cr.astype(jnp.float32).reshape(b, n_c, 64)

    a = _A_NP
    m1y = jnp.asarray(np.kron(np.eye(width // 8), a), dtype=jnp.float32)
    m2y = jnp.asarray(np.kron(np.eye(tile_h // 8), a.T), dtype=jnp.float32)
    a_up_cols = np.repeat(a, 2, axis=1)
    a_up_rows = np.repeat(a.T, 2, axis=0)
    m1c = jnp.asarray(np.kron(np.eye(width // 16), a_up_cols),
                      dtype=jnp.float32)
    m2c = jnp.asarray(np.kron(np.eye(tile_h // 16), a_up_rows),
                      dtype=jnp.float32)

    qy = jnp.tile(y_qt.astype(jnp.float32), (tile_h // 8, width // 8))
    qc = jnp.tile(c_qt.astype(jnp.float32), (tile_h // 16, width // 16))

    cst = lambda r, c: pl.BlockSpec((r, c), lambda bi, i: (0, 0))
    return pl.pallas_call(
        functools.partial(_dec_kernel, bry=bry, brc=brc, wblk=wblk,
                          cblk=cblk),
        out_shape=jax.ShapeDtypeStruct((b, 3, height, width), jnp.float32),
        grid=(b, ntiles),
        in_specs=[
            pl.BlockSpec((None, bry * wblk, 64), lambda bi, i: (bi, i, 0)),
            pl.BlockSpec((None, brc * cblk, 64), lambda bi, i: (bi, i, 0)),
            pl.BlockSpec((None, brc * cblk, 64), lambda bi, i: (bi, i, 0)),
            cst(tile_h, width), cst(tile_h // 2, width // 2),
            cst(tile_h, tile_h), cst(width, width),
            cst(tile_h, tile_h // 2), cst(width // 2, width),
        ],
        out_specs=pl.BlockSpec((None, 3, tile_h, width),
                               lambda bi, i: (bi, 0, i, 0)),
        compiler_params=pltpu.CompilerParams(
            dimension_semantics=("parallel", "arbitrary")),
    )(y2, cb2, cr2, qy, qc, m2y, m1y, m2c, m1c)


def kernel(y, cb, cr, y_qt, c_qt):
    return _decompress(y, cb, cr, y_qt, c_qt, 512, 512)


# speedup vs baseline: 1.3520x; 1.0638x over previous
import functools

import numpy as np
import jax
import jax.numpy as jnp
from jax.experimental import pallas as pl
from jax.experimental.pallas import tpu as pltpu


def _dct_factors():
    alpha = np.array([1.0 / np.sqrt(2)] + [1.0] * 7, dtype=np.float64)
    k = np.arange(8)
    cos = np.cos((2 * k[None, :] + 1) * k[:, None] * np.pi / 16)
    return 0.5 * alpha[:, None] * cos


_A_NP = _dct_factors()


def _merge(x, nbr, nbc):
    xb = x.astype(jnp.bfloat16)
    m = (xb.reshape(nbr, nbc, 8, 8).transpose(0, 2, 1, 3)
         .reshape(nbr * 8, nbc * 8))
    return m.astype(jnp.float32)


def _dec_kernel(y_ref, cb_ref, cr_ref, qy_ref, qc_ref,
                m2y_ref, m1y_ref, m2c_ref, m1c_ref, o_ref, *,
                bry, brc, wblk, cblk):
    cy = _merge(y_ref[...], bry, wblk) * qy_ref[...]
    t = jnp.dot(cy, m1y_ref[...], preferred_element_type=jnp.float32)
    yimg = jnp.dot(m2y_ref[...], t, preferred_element_type=jnp.float32) + 128.0

    ccb = _merge(cb_ref[...], brc, cblk) * qc_ref[...]
    ccr = _merge(cr_ref[...], brc, cblk) * qc_ref[...]
    tcb = jnp.dot(m2c_ref[...], ccb, preferred_element_type=jnp.float32)
    tcr = jnp.dot(m2c_ref[...], ccr, preferred_element_type=jnp.float32)
    cb2 = jnp.dot(tcb, m1c_ref[...], preferred_element_type=jnp.float32)
    cr2 = jnp.dot(tcr, m1c_ref[...], preferred_element_type=jnp.float32)

    o_ref[0] = jnp.clip(yimg + 1.402 * cr2, 0.0, 255.0)
    o_ref[1] = jnp.clip(yimg - 0.344136 * cb2 - 0.714136 * cr2, 0.0, 255.0)
    o_ref[2] = jnp.clip(yimg + 1.772 * cb2, 0.0, 255.0)


def _decompress(y, cb, cr, y_qt, c_qt, height, width):
    b, n_y = y.shape[0], y.shape[1]
    n_c = cb.shape[1]
    tile_h = 128 if height % 128 == 0 else height
    ntiles = height // tile_h
    wblk, cblk = width // 8, width // 16
    bry, brc = tile_h // 8, tile_h // 16

    y2 = y.astype(jnp.float32).reshape(b, n_y, 64)
    cb2 = cb.astype(jnp.float32).reshape(b, n_c, 64)
    cr2 = cr.astype(jnp.float32).reshape(b, n_c, 64)

    a = _A_NP
    m1y = jnp.asarray(np.kron(np.eye(width // 8), a), dtype=jnp.float32)
    m2y = jnp.asarray(np.kron(np.eye(tile_h // 8), a.T), dtype=jnp.float32)
    a_up_cols = np.repeat(a, 2, axis=1)
    a_up_rows = np.repeat(a.T, 2, axis=0)
    m1c = jnp.asarray(np.kron(np.eye(width // 16), a_up_cols),
                      dtype=jnp.float32)
    m2c = jnp.asarray(np.kron(np.eye(tile_h // 16), a_up_rows),
                      dtype=jnp.float32)

    qy = jnp.tile(y_qt.astype(jnp.float32), (tile_h // 8, width // 8))
    qc = jnp.tile(c_qt.astype(jnp.float32), (tile_h // 16, width // 16))

    cst = lambda r, c: pl.BlockSpec((r, c), lambda bi, i: (0, 0))
    return pl.pallas_call(
        functools.partial(_dec_kernel, bry=bry, brc=brc, wblk=wblk,
                          cblk=cblk),
        out_shape=jax.ShapeDtypeStruct((b, 3, height, width), jnp.float32),
        grid=(b, ntiles),
        in_specs=[
            pl.BlockSpec((None, bry * wblk, 64), lambda bi, i: (bi, i, 0)),
            pl.BlockSpec((None, brc * cblk, 64), lambda bi, i: (bi, i, 0)),
            pl.BlockSpec((None, brc * cblk, 64), lambda bi, i: (bi, i, 0)),
            cst(tile_h, width), cst(tile_h // 2, width // 2),
            cst(tile_h, tile_h), cst(width, width),
            cst(tile_h, tile_h // 2), cst(width // 2, width),
        ],
        out_specs=pl.BlockSpec((None, 3, tile_h, width),
                               lambda bi, i: (bi, 0, i, 0)),
        compiler_params=pltpu.CompilerParams(
            dimension_semantics=("parallel", "arbitrary")),
    )(y2, cb2, cr2, qy, qc, m2y, m1y, m2c, m1c)


def kernel(y, cb, cr, y_qt, c_qt):
    return _decompress(y, cb, cr, y_qt, c_qt, 512, 512)
